# Initial kernel scaffold; baseline (speedup 1.0000x reference)
import jax
import jax.numpy as jnp
from jax.experimental import pallas as pl


def kernel(x, token_table, pos_table):
    raise NotImplementedError("write your pallas kernel here")


# speedup vs baseline: 2.6478x; 2.6478x over previous
import functools

import jax
import jax.numpy as jnp
from jax import lax
from jax.experimental import pallas as pl
from jax.experimental.pallas import tpu as pltpu
from jax.experimental.pallas import tpu_sc as plsc

MAXLEN = 200
VOCAB = 100000
EMBED = 64
BATCH = 4096

NC, NS = 2, 16
NW = NC * NS
NROWS = BATCH * MAXLEN
ROWS_PER_W = NROWS // NW
BLK_SEQ = 2
BLK = BLK_SEQ * MAXLEN
NBLK = ROWS_PER_W // BLK
CHUNKS = ((0, 128), (128, 128), (256, 128), (384, 16))
ADD_UNROLL = 8

_mesh = plsc.VectorSubcoreMesh(core_axis_name="c", subcore_axis_name="s")


@functools.partial(
    pl.kernel,
    out_type=jax.ShapeDtypeStruct((NROWS, EMBED), jnp.float32),
    mesh=_mesh,
    compiler_params=pltpu.CompilerParams(use_tc_tiling_on_sc=False),
    scratch_types=[
        pltpu.VMEM((BLK, EMBED), jnp.float32),
        pltpu.VMEM((BLK, EMBED), jnp.float32),
        pltpu.VMEM((BLK, EMBED), jnp.float32),
        pltpu.VMEM((BLK,), jnp.int32),
        pltpu.VMEM((BLK,), jnp.int32),
        pltpu.SemaphoreType.DMA,
        pltpu.SemaphoreType.DMA,
        pltpu.SemaphoreType.DMA,
        pltpu.SemaphoreType.DMA,
    ],
)
def _sc_embed(idx_hbm, tok_hbm, pos_hbm, out_hbm,
              posrep, rows0, rows1, idx0, idx1,
              gsem0, gsem1, wsem0, wsem1):
    wid = lax.axis_index("s") * NC + lax.axis_index("c")
    base = wid * ROWS_PER_W
    rows = (rows0, rows1)
    idxb = (idx0, idx1)
    gsem = (gsem0, gsem1)
    wsem = (wsem0, wsem1)

    pltpu.sync_copy(pos_hbm, posrep.at[pl.ds(0, MAXLEN)])
    pltpu.sync_copy(pos_hbm, posrep.at[pl.ds(MAXLEN, MAXLEN)])

    def start_gathers(g, p):
        pltpu.sync_copy(idx_hbm.at[pl.ds(base + g * BLK, BLK)], idxb[p])
        for off, n in CHUNKS:
            pltpu.async_copy(
                tok_hbm.at[idxb[p].at[pl.ds(off, n)]],
                rows[p].at[pl.ds(off, n)],
                gsem[p],
            )

    def drain_gathers(p):
        pltpu.make_async_copy(tok_hbm.at[pl.ds(0, BLK)], rows[p], gsem[p]).wait()

    def drain_write(p):
        pltpu.make_async_copy(rows[p], out_hbm.at[pl.ds(0, BLK)], wsem[p]).wait()

    start_gathers(0, 0)

    @pl.loop(0, NBLK // 2)
    def _pipeline(h):
        for b in (0, 1):
            g = h * 2 + b
            p, q = b, 1 - b

            @pl.when(g + 1 < NBLK)
            def _():
                @pl.when(g >= 1)
                def _():
                    drain_write(q)
                start_gathers(g + 1, q)

            drain_gathers(p)

            @pl.loop(0, BLK, unroll=ADD_UNROLL)
            def _add(r):
                for c4 in range(EMBED // 16):
                    sl = pl.ds(c4 * 16, 16)
                    rows[p][r, sl] += posrep[r, sl]

            pltpu.async_copy(rows[p], out_hbm.at[pl.ds(base + g * BLK, BLK)],
                             wsem[p])

    drain_write(0)
    drain_write(1)


def kernel(x, token_table, pos_table):
    idx = x.reshape(-1).astype(jnp.int32)
    out = _sc_embed(idx, token_table, pos_table)
    return out.reshape(BATCH, MAXLEN, EMBED)


# speedup vs baseline: 3.8024x; 1.4360x over previous
import functools

import jax
import jax.numpy as jnp
from jax import lax
from jax.experimental import pallas as pl
from jax.experimental.pallas import tpu as pltpu
from jax.experimental.pallas import tpu_sc as plsc

MAXLEN = 200
VOCAB = 100000
EMBED = 64
BATCH = 4096

NC, NS = 2, 16
NW = NC * NS
SEQ_PER_W = BATCH // NW
BLK_SEQ = 2
BLK = BLK_SEQ * MAXLEN
NBLK = SEQ_PER_W // BLK_SEQ
SEQ_CHUNKS = ((0, 128), (128, 72))
ADD_UNROLL = 8

_mesh = plsc.VectorSubcoreMesh(core_axis_name="c", subcore_axis_name="s")


@functools.partial(
    pl.kernel,
    out_type=jax.ShapeDtypeStruct((BATCH, MAXLEN, EMBED), jnp.float32),
    mesh=_mesh,
    compiler_params=pltpu.CompilerParams(use_tc_tiling_on_sc=False),
    scratch_types=[
        pltpu.VMEM((MAXLEN, EMBED), jnp.float32),
        pltpu.VMEM((BLK_SEQ, MAXLEN, EMBED), jnp.float32),
        pltpu.VMEM((BLK_SEQ, MAXLEN, EMBED), jnp.float32),
        pltpu.VMEM((BLK_SEQ, MAXLEN), jnp.int32),
        pltpu.VMEM((BLK_SEQ, MAXLEN), jnp.int32),
        pltpu.SemaphoreType.DMA,
        pltpu.SemaphoreType.DMA,
        pltpu.SemaphoreType.DMA,
        pltpu.SemaphoreType.DMA,
    ],
)
def _sc_embed(idx_hbm, tok_hbm, pos_hbm, out_hbm,
              pos_v, rows0, rows1, idx0, idx1,
              gsem0, gsem1, wsem0, wsem1):
    wid = lax.axis_index("s") * NC + lax.axis_index("c")
    seq_base = wid * SEQ_PER_W
    rows = (rows0, rows1)
    idxb = (idx0, idx1)
    gsem = (gsem0, gsem1)
    wsem = (wsem0, wsem1)

    pltpu.sync_copy(pos_hbm, pos_v)

    def start_gathers(g, p):
        pltpu.sync_copy(idx_hbm.at[pl.ds(seq_base + g * BLK_SEQ, BLK_SEQ)],
                        idxb[p])
        for s in range(BLK_SEQ):
            for off, n in SEQ_CHUNKS:
                pltpu.async_copy(
                    tok_hbm.at[idxb[p].at[s, pl.ds(off, n)]],
                    rows[p].at[s, pl.ds(off, n)],
                    gsem[p],
                )

    def drain_gathers(p):
        for s in range(BLK_SEQ):
            for off, n in SEQ_CHUNKS:
                pltpu.make_async_copy(tok_hbm.at[pl.ds(0, n)],
                                      rows[p].at[s, pl.ds(off, n)],
                                      gsem[p]).wait()

    def drain_write(p):
        pltpu.make_async_copy(rows[p], out_hbm.at[pl.ds(0, BLK_SEQ)],
                              wsem[p]).wait()

    start_gathers(0, 0)

    @pl.loop(0, NBLK // 2)
    def _pipeline(h):
        for b in (0, 1):
            g = h * 2 + b
            p, q = b, 1 - b

            @pl.when(g + 1 < NBLK)
            def _():
                @pl.when(g >= 1)
                def _():
                    drain_write(q)
                start_gathers(g + 1, q)

            drain_gathers(p)

            for s in range(BLK_SEQ):
                @plsc.parallel_loop(0, MAXLEN, unroll=ADD_UNROLL)
                def _add(r):
                    for c4 in range(EMBED // 16):
                        sl = pl.ds(c4 * 16, 16)
                        rows[p][s, r, sl] += pos_v[r, sl]

            pltpu.async_copy(
                rows[p],
                out_hbm.at[pl.ds(seq_base + g * BLK_SEQ, BLK_SEQ)],
                wsem[p])

    drain_write(0)
    drain_write(1)


def kernel(x, token_table, pos_table):
    return _sc_embed(x.astype(jnp.int32), token_table, pos_table)


# speedup vs baseline: 4.4368x; 1.1669x over previous
import functools

import jax
import jax.numpy as jnp
from jax import lax
from jax.experimental import pallas as pl
from jax.experimental.pallas import tpu as pltpu
from jax.experimental.pallas import tpu_sc as plsc

MAXLEN = 200
VOCAB = 100000
EMBED = 64
BATCH = 4096
LANES = 128

NC, NS = 2, 16
NW = NC * NS
SEQ_PER_W = BATCH // NW
CHUNK0 = (0, 128)
CHUNK1 = (128, 72)
ADD_UNROLL = 8

_mesh = plsc.VectorSubcoreMesh(core_axis_name="c", subcore_axis_name="s")


@functools.partial(
    pl.kernel,
    out_type=jax.ShapeDtypeStruct((BATCH, MAXLEN, EMBED), jnp.float32),
    mesh=_mesh,
    compiler_params=pltpu.CompilerParams(use_tc_tiling_on_sc=True),
    scratch_types=[
        pltpu.VMEM((SEQ_PER_W, MAXLEN), jnp.int32),
        pltpu.VMEM((MAXLEN, EMBED), jnp.float32),
        pltpu.VMEM((CHUNK0[1], LANES), jnp.float32),
        pltpu.VMEM((CHUNK1[1], LANES), jnp.float32),
        pltpu.VMEM((CHUNK0[1], EMBED), jnp.float32),
        pltpu.VMEM((CHUNK1[1], EMBED), jnp.float32),
        pltpu.SemaphoreType.DMA,
        pltpu.SemaphoreType.DMA,
        pltpu.SemaphoreType.DMA,
        pltpu.SemaphoreType.DMA,
    ],
)
def _sc_embed(idx_hbm, tok_hbm, pos_hbm, out_hbm,
              idx_all, pos_v, rows0, rows1, stg0, stg1,
              gsem0, gsem1, wsem0, wsem1):
    wid = lax.axis_index("s") * NC + lax.axis_index("c")
    seq_base = wid * SEQ_PER_W
    rows = (rows0, rows1)
    stg = (stg0, stg1)
    gsem = (gsem0, gsem1)
    wsem = (wsem0, wsem1)
    chunk = (CHUNK0, CHUNK1)

    pltpu.sync_copy(idx_hbm.at[pl.ds(seq_base, SEQ_PER_W)], idx_all)
    pltpu.sync_copy(pos_hbm, pos_v)

    def start_gather(s, p):
        off, n = chunk[p]
        pltpu.async_copy(
            tok_hbm.at[idx_all.at[s, pl.ds(off, n)]], rows[p], gsem[p])

    def drain_gather(p):
        _, n = chunk[p]
        pltpu.make_async_copy(tok_hbm.at[pl.ds(0, n)], rows[p],
                              gsem[p]).wait()

    def add_and_write(s, p):
        off, n = chunk[p]

        @plsc.parallel_loop(0, n, unroll=ADD_UNROLL)
        def _add(r):
            for c4 in range(EMBED // 16):
                sl = pl.ds(c4 * 16, 16)
                stg[p][r, sl] = rows[p][r, sl] + pos_v[off + r, sl]

        pltpu.async_copy(stg[p], out_hbm.at[seq_base + s, pl.ds(off, n)],
                         wsem[p])

    def drain_write(p):
        off, n = chunk[p]
        pltpu.make_async_copy(stg[p], out_hbm.at[0, pl.ds(off, n)],
                              wsem[p]).wait()

    start_gather(0, 0)

    @pl.loop(0, SEQ_PER_W)
    def _pipeline(h):
        start_gather(h, 1)
        drain_gather(0)

        @pl.when(h >= 1)
        def _():
            drain_write(0)
        add_and_write(h, 0)

        @pl.when(h + 1 < SEQ_PER_W)
        def _():
            start_gather(h + 1, 0)
        drain_gather(1)

        @pl.when(h >= 1)
        def _():
            drain_write(1)
        add_and_write(h, 1)

    drain_write(0)
    drain_write(1)


def kernel(x, token_table, pos_table):
    tok_padded = jnp.pad(token_table, ((0, 0), (0, LANES - EMBED)))
    return _sc_embed(x.astype(jnp.int32), tok_padded, pos_table)
